# Initial kernel scaffold; baseline (speedup 1.0000x reference)
import jax
import jax.numpy as jnp
from jax.experimental import pallas as pl


def kernel(h, edge_index, WQ, WK, WV):
    raise NotImplementedError("write your pallas kernel here")


# speedup vs baseline: 13.2052x; 13.2052x over previous
import functools
import jax
import jax.numpy as jnp
from jax import lax
from jax.experimental import pallas as pl
from jax.experimental.pallas import tpu as pltpu
from jax.experimental.pallas import tpu_sc as plsc

H = 8
D = 16
HD = H * D
ROW = HD + 16
INV_SQRT_D = 0.25

NC = 2
NS = 16
NW = NC * NS
C = 32


def _cdiv(a, b):
    return (a + b - 1) // b


def _qkv_body(h_ref, wt_ref, q_ref, kv_ref):
    y = jnp.dot(h_ref[...], wt_ref[...], preferred_element_type=jnp.float32)
    q_ref[...] = y[:, :HD]
    kv_ref[...] = y[:, HD:]


def _qkv(h, wt, blk):
    n = h.shape[0]
    in_dim = h.shape[1]
    grid = n // blk
    return pl.pallas_call(
        _qkv_body,
        grid=(grid,),
        in_specs=[
            pl.BlockSpec((blk, in_dim), lambda i: (i, 0)),
            pl.BlockSpec((in_dim, 3 * HD), lambda i: (0, 0)),
        ],
        out_specs=[
            pl.BlockSpec((blk, HD), lambda i: (i, 0)),
            pl.BlockSpec((blk, 2 * HD), lambda i: (i, 0)),
        ],
        out_shape=[
            jax.ShapeDtypeStruct((n, HD), jnp.float32),
            jax.ShapeDtypeStruct((n, 2 * HD), jnp.float32),
        ],
    )(h, wt)


def _edge_body(acc_rows, chunks_per_worker,
               q_hbm, kv_hbm, src_hbm, dst_hbm, out_hbm,
               acc_s, src_v, dst_v, kvbuf, qbuf, wbuf, zb, sem_kv, sem_q):
    c = lax.axis_index("c")
    s = lax.axis_index("s")
    w = s * NC + c

    zb_rows = zb.shape[0]
    rpt = acc_rows // NS

    def _zero_row(i, carry):
        for j in range(ROW // 16):
            zb[i, pl.ds(j * 16, 16)] = jnp.zeros((16,), jnp.float32)
        return carry
    lax.fori_loop(0, zb_rows, _zero_row, 0)

    r0 = s * rpt

    def _zcopy(t, carry):
        pltpu.sync_copy(zb, acc_s.at[pl.ds(r0 + t * zb_rows, zb_rows)])
        return carry
    lax.fori_loop(0, rpt // zb_rows, _zcopy, 0)

    for r in range(C):
        wbuf[r, pl.ds(HD, 16)] = jnp.zeros((16,), jnp.float32)

    plsc.subcore_barrier()

    ebase = w * (chunks_per_worker * C)

    def _chunk(j, carry):
        b = ebase + j * C
        pltpu.sync_copy(src_hbm.at[pl.ds(b, C)], src_v)
        pltpu.sync_copy(dst_hbm.at[pl.ds(b, C)], dst_v)
        cp_kv = pltpu.async_copy(kv_hbm.at[src_v], kvbuf, sem_kv)
        cp_q = pltpu.async_copy(q_hbm.at[dst_v], qbuf, sem_q)
        cp_kv.wait()
        cp_q.wait()
        for g in range(C // 16):
            rows = lax.iota(jnp.int32, 16) + (g * 16)
            for hh in range(H):
                col0 = hh * D
                acc = None
                for d in range(D):
                    ci = jnp.full((16,), col0 + d, jnp.int32)
                    kvv = plsc.load_gather(kvbuf, [rows, ci])
                    qv = plsc.load_gather(qbuf, [rows, ci])
                    prod = kvv * qv
                    acc = prod if acc is None else acc + prod
                sc = acc * INV_SQRT_D
                sc = jnp.minimum(jnp.maximum(sc, -5.0), 5.0)
                p = jnp.exp(sc)
                plsc.store_scatter(
                    wbuf, [rows, jnp.full((16,), HD + hh, jnp.int32)], p)
                for d in range(D):
                    vi = jnp.full((16,), HD + col0 + d, jnp.int32)
                    vv = plsc.load_gather(kvbuf, [rows, vi])
                    plsc.store_scatter(
                        wbuf, [rows, jnp.full((16,), col0 + d, jnp.int32)],
                        vv * p)
        pltpu.sync_copy(wbuf, acc_s.at[dst_v], add=True)
        return carry

    lax.fori_loop(0, chunks_per_worker, _chunk, 0)

    plsc.subcore_barrier()

    pltpu.sync_copy(acc_s.at[pl.ds(r0, rpt)], out_hbm.at[c, pl.ds(r0, rpt)])


def _edge_phase(q, kv, src, dst, acc_rows, chunks_per_worker):
    mesh = plsc.VectorSubcoreMesh(core_axis_name="c", subcore_axis_name="s")
    zb_rows = 16
    body = functools.partial(_edge_body, acc_rows, chunks_per_worker)
    return pl.kernel(
        body,
        out_type=jax.ShapeDtypeStruct((NC, acc_rows, ROW), jnp.float32),
        mesh=mesh,
        scratch_types=[
            pltpu.VMEM_SHARED((acc_rows, ROW), jnp.float32),
            pltpu.VMEM((C,), jnp.int32),
            pltpu.VMEM((C,), jnp.int32),
            pltpu.VMEM((C, 2 * HD), jnp.float32),
            pltpu.VMEM((C, HD), jnp.float32),
            pltpu.VMEM((C, ROW), jnp.float32),
            pltpu.VMEM((zb_rows, ROW), jnp.float32),
            pltpu.SemaphoreType.DMA,
            pltpu.SemaphoreType.DMA,
        ],
        compiler_params=pltpu.CompilerParams(
            use_tc_tiling_on_sc=False, needs_layout_passes=False),
    )(q, kv, src, dst)


def _combine_body(acc_ref, out_ref):
    ab = acc_ref[...]
    a = ab[0] + ab[1]
    wv = a[:, :HD]
    z = a[:, HD:HD + H]
    rowi = lax.broadcasted_iota(jnp.int32, (H, HD), 0)
    coli = lax.broadcasted_iota(jnp.int32, (H, HD), 1)
    bmat = (coli // D == rowi).astype(jnp.float32)
    zrep = jnp.dot(z, bmat, preferred_element_type=jnp.float32)
    out_ref[...] = wv / zrep


def _combine(acc, n_out, blk):
    acc_rows = acc.shape[1]
    grid = n_out // blk
    return pl.pallas_call(
        _combine_body,
        grid=(grid,),
        in_specs=[pl.BlockSpec((NC, blk, ROW), lambda i: (0, i, 0))],
        out_specs=pl.BlockSpec((blk, HD), lambda i: (i, 0)),
        out_shape=jax.ShapeDtypeStruct((n_out, HD), jnp.float32),
    )(acc)


def kernel(h, edge_index, WQ, WK, WV):
    n, in_dim = h.shape
    e = edge_index.shape[1]

    wt = jnp.concatenate([WQ.T, WK.T, WV.T], axis=1)
    blk_n = 1000 if n % 1000 == 0 else 8
    n_pad = _cdiv(n, blk_n) * blk_n
    h_p = h if n_pad == n else jnp.pad(h, ((0, n_pad - n), (0, 0)))

    src = edge_index[0]
    dst = edge_index[1]
    epw = NW * C
    chunks_per_worker = _cdiv(e, epw)
    e_pad = chunks_per_worker * epw
    if e_pad != e:
        src = jnp.concatenate(
            [src, jnp.zeros((e_pad - e,), jnp.int32)])
        dst = jnp.concatenate(
            [dst, jnp.full((e_pad - e,), n, jnp.int32)])

    acc_rows = max(n + 1, _cdiv(n, 1000) * 1000)
    acc_rows = 16 * NS * _cdiv(acc_rows, 16 * NS)

    q, kv = _qkv(h_p, wt, blk_n)
    q = q[:n] if n_pad != n else q
    kv = kv[:n] if n_pad != n else kv
    acc = _edge_phase(q, kv, src, dst, acc_rows, chunks_per_worker)
    blk_o = 1000 if n % 1000 == 0 else 8
    n_out = _cdiv(n, blk_o) * blk_o
    out = _combine(acc, n_out, blk_o)
    return out[:n].reshape(n, H, D)


# speedup vs baseline: 14.6791x; 1.1116x over previous
import functools
import jax
import jax.numpy as jnp
from jax import lax
from jax.experimental import pallas as pl
from jax.experimental.pallas import tpu as pltpu
from jax.experimental.pallas import tpu_sc as plsc

H = 8
D = 16
HD = H * D
ROW = HD + H
INV_SQRT_D = 0.25

NC = 2
NS = 16
NW = NC * NS
C = 16


def _cdiv(a, b):
    return (a + b - 1) // b


def _qkv_body(h_ref, wt_ref, q_ref, kv_ref):
    y = jnp.dot(h_ref[...], wt_ref[...], preferred_element_type=jnp.float32)
    q_ref[...] = y[:, :HD]
    kv_ref[...] = y[:, HD:]


def _qkv(h, wt, blk):
    n = h.shape[0]
    in_dim = h.shape[1]
    grid = n // blk
    return pl.pallas_call(
        _qkv_body,
        grid=(grid,),
        in_specs=[
            pl.BlockSpec((blk, in_dim), lambda i: (i, 0)),
            pl.BlockSpec((in_dim, 3 * HD), lambda i: (0, 0)),
        ],
        out_specs=[
            pl.BlockSpec((blk, HD), lambda i: (i, 0)),
            pl.BlockSpec((blk, 2 * HD), lambda i: (i, 0)),
        ],
        out_shape=[
            jax.ShapeDtypeStruct((n, HD), jnp.float32),
            jax.ShapeDtypeStruct((n, 2 * HD), jnp.float32),
        ],
    )(h, wt)


def _compute_chunk(kv_b, q_b, w_b):
    rows = lax.iota(jnp.int32, 16)
    for hh in range(H):
        col0 = hh * D
        acc = None
        for d in range(D):
            ci = jnp.full((16,), col0 + d, jnp.int32)
            kvv = plsc.load_gather(kv_b, [rows, ci])
            qv = plsc.load_gather(q_b, [rows, ci])
            prod = kvv * qv
            acc = prod if acc is None else acc + prod
        sc = acc * INV_SQRT_D
        sc = jnp.minimum(jnp.maximum(sc, -5.0), 5.0)
        p = jnp.exp(sc)
        plsc.store_scatter(
            w_b, [rows, jnp.full((16,), HD + hh, jnp.int32)], p)
        for d in range(D):
            vi = jnp.full((16,), HD + col0 + d, jnp.int32)
            vv = plsc.load_gather(kv_b, [rows, vi])
            plsc.store_scatter(
                w_b, [rows, jnp.full((16,), col0 + d, jnp.int32)],
                vv * p)


def _edge_body(acc_rows, ch,
               q_hbm, kv_hbm, src_hbm, dst_hbm, zeros_hbm, out_hbm,
               acc_s, srcblk, dstblk, kvbuf, qbuf, wbuf,
               sem_kv0, sem_kv1, sem_q0, sem_q1):
    c = lax.axis_index("c")
    s = lax.axis_index("s")
    w = s * NC + c

    sem_kv = (sem_kv0, sem_kv1)
    sem_q = (sem_q0, sem_q1)

    rpt = acc_rows // NS
    r0 = s * rpt
    pltpu.sync_copy(zeros_hbm.at[pl.ds(r0, rpt)], acc_s.at[pl.ds(r0, rpt)])

    row0 = w * ch
    pltpu.sync_copy(src_hbm.at[pl.ds(row0, ch + 2)], srcblk)
    pltpu.sync_copy(dst_hbm.at[pl.ds(row0, ch + 2)], dstblk)

    plsc.subcore_barrier()

    def _issue(slot, local_row):
        pltpu.async_copy(
            kv_hbm.at[srcblk.at[local_row]], kvbuf.at[slot], sem_kv[slot])
        pltpu.async_copy(
            q_hbm.at[dstblk.at[local_row]], qbuf.at[slot], sem_q[slot])

    def _drain(slot):
        pltpu.make_async_copy(
            kv_hbm.at[srcblk.at[0]], kvbuf.at[slot], sem_kv[slot]).wait()
        pltpu.make_async_copy(
            q_hbm.at[dstblk.at[0]], qbuf.at[slot], sem_q[slot]).wait()

    _issue(0, 0)
    _issue(1, 1)

    def _pair(qq, carry):
        jj = 2 * qq
        for b in range(2):
            _drain(b)
            _compute_chunk(kvbuf.at[b], qbuf.at[b], wbuf.at[b])
            pltpu.sync_copy(
                wbuf.at[b], acc_s.at[dstblk.at[jj + b]], add=True)
            _issue(b, jj + b + 2)
        return carry

    lax.fori_loop(0, ch // 2, _pair, 0)

    _drain(0)
    _drain(1)

    plsc.subcore_barrier()
    pltpu.sync_copy(acc_s.at[pl.ds(r0, rpt)], out_hbm.at[c, pl.ds(r0, rpt)])


def _edge_phase(q, kv, src2, dst2, zeros, acc_rows, ch):
    mesh = plsc.VectorSubcoreMesh(core_axis_name="c", subcore_axis_name="s")
    body = functools.partial(_edge_body, acc_rows, ch)
    return pl.kernel(
        body,
        out_type=jax.ShapeDtypeStruct((NC, acc_rows, ROW), jnp.float32),
        mesh=mesh,
        scratch_types=[
            pltpu.VMEM_SHARED((acc_rows, ROW), jnp.float32),
            pltpu.VMEM((ch + 2, C), jnp.int32),
            pltpu.VMEM((ch + 2, C), jnp.int32),
            pltpu.VMEM((2, C, 2 * HD), jnp.float32),
            pltpu.VMEM((2, C, HD), jnp.float32),
            pltpu.VMEM((2, C, ROW), jnp.float32),
            pltpu.SemaphoreType.DMA,
            pltpu.SemaphoreType.DMA,
            pltpu.SemaphoreType.DMA,
            pltpu.SemaphoreType.DMA,
        ],
        compiler_params=pltpu.CompilerParams(
            use_tc_tiling_on_sc=False, needs_layout_passes=False),
    )(q, kv, src2, dst2, zeros)


def _combine_body(acc_ref, out_ref):
    ab = acc_ref[...]
    a = ab[0] + ab[1]
    wv = a[:, :HD]
    z = a[:, HD:HD + H]
    rowi = lax.broadcasted_iota(jnp.int32, (H, HD), 0)
    coli = lax.broadcasted_iota(jnp.int32, (H, HD), 1)
    bmat = (coli // D == rowi).astype(jnp.float32)
    zrep = jnp.dot(z, bmat, preferred_element_type=jnp.float32)
    out_ref[...] = wv / zrep


def _combine(acc, n_out, blk):
    grid = n_out // blk
    return pl.pallas_call(
        _combine_body,
        grid=(grid,),
        in_specs=[pl.BlockSpec((NC, blk, ROW), lambda i: (0, i, 0))],
        out_specs=pl.BlockSpec((blk, HD), lambda i: (i, 0)),
        out_shape=jax.ShapeDtypeStruct((n_out, HD), jnp.float32),
    )(acc)


def kernel(h, edge_index, WQ, WK, WV):
    n, in_dim = h.shape
    e = edge_index.shape[1]

    wt = jnp.concatenate([WQ.T, WK.T, WV.T], axis=1)
    blk_n = 1000 if n % 1000 == 0 else 8
    n_pad = _cdiv(n, blk_n) * blk_n
    h_p = h if n_pad == n else jnp.pad(h, ((0, n_pad - n), (0, 0)))

    src = edge_index[0]
    dst = edge_index[1]
    ch = 2 * _cdiv(e, NW * C * 2)
    rows_total = NW * ch + 2
    e_pad = rows_total * C
    src = jnp.concatenate([src, jnp.zeros((e_pad - e,), jnp.int32)])
    dst = jnp.concatenate([dst, jnp.full((e_pad - e,), n, jnp.int32)])
    src2 = src.reshape(rows_total, C)
    dst2 = dst.reshape(rows_total, C)

    acc_rows = max(n + 1, _cdiv(n, 1000) * 1000)
    acc_rows = NS * _cdiv(acc_rows, NS)
    zeros = jnp.zeros((acc_rows, ROW), jnp.float32)

    q, kv = _qkv(h_p, wt, blk_n)
    q = q[:n] if n_pad != n else q
    kv = kv[:n] if n_pad != n else kv
    acc = _edge_phase(q, kv, src2, dst2, zeros, acc_rows, ch)
    blk_o = 1000 if n % 1000 == 0 else 8
    n_out = _cdiv(n, blk_o) * blk_o
    out = _combine(acc, n_out, blk_o)
    return out[:n].reshape(n, H, D)


# speedup vs baseline: 15.0928x; 1.0282x over previous
import functools
import jax
import jax.numpy as jnp
from jax import lax
from jax.experimental import pallas as pl
from jax.experimental.pallas import tpu as pltpu
from jax.experimental.pallas import tpu_sc as plsc

H = 8
D = 16
HD = H * D
ROW = HD + H
INV_SQRT_D = 0.25

NC = 2
NS = 16
NW = NC * NS
C = 16


def _cdiv(a, b):
    return (a + b - 1) // b


def _qkv_body(h_ref, wt_ref, q_ref, kv_ref):
    y = jnp.dot(h_ref[...], wt_ref[...], preferred_element_type=jnp.float32)
    q_ref[...] = y[:, :HD]
    kv_ref[...] = y[:, HD:]


def _qkv(h, wt, blk):
    n = h.shape[0]
    in_dim = h.shape[1]
    grid = n // blk
    return pl.pallas_call(
        _qkv_body,
        grid=(grid,),
        in_specs=[
            pl.BlockSpec((blk, in_dim), lambda i: (i, 0)),
            pl.BlockSpec((in_dim, 3 * HD), lambda i: (0, 0)),
        ],
        out_specs=[
            pl.BlockSpec((blk, HD), lambda i: (i, 0)),
            pl.BlockSpec((blk, 2 * HD), lambda i: (i, 0)),
        ],
        out_shape=[
            jax.ShapeDtypeStruct((n, HD), jnp.float32),
            jax.ShapeDtypeStruct((n, 2 * HD), jnp.float32),
        ],
    )(h, wt)


def _compute_chunk(kv_b, q_b, w_b):
    rows = lax.iota(jnp.int32, 16)
    for hh in range(H):
        col0 = hh * D
        acc = None
        for d in range(D):
            ci = jnp.full((16,), col0 + d, jnp.int32)
            kvv = plsc.load_gather(kv_b, [rows, ci])
            qv = plsc.load_gather(q_b, [rows, ci])
            prod = kvv * qv
            acc = prod if acc is None else acc + prod
        sc = acc * INV_SQRT_D
        sc = jnp.minimum(jnp.maximum(sc, -5.0), 5.0)
        p = jnp.exp(sc)
        plsc.store_scatter(
            w_b, [rows, jnp.full((16,), HD + hh, jnp.int32)], p)
        for d in range(D):
            vi = jnp.full((16,), HD + col0 + d, jnp.int32)
            vv = plsc.load_gather(kv_b, [rows, vi])
            plsc.store_scatter(
                w_b, [rows, jnp.full((16,), col0 + d, jnp.int32)],
                vv * p)


def _edge_body(acc_rows, ch,
               q_hbm, kv_hbm, src_hbm, dst_hbm, zeros_hbm, out_hbm,
               acc_s, srcblk, dstblk, kvbuf, qbuf, wbuf,
               sem_kv0, sem_kv1, sem_q0, sem_q1, sem_w0, sem_w1):
    c = lax.axis_index("c")
    s = lax.axis_index("s")
    w = s * NC + c

    sem_kv = (sem_kv0, sem_kv1)
    sem_q = (sem_q0, sem_q1)
    sem_w = (sem_w0, sem_w1)

    rpt = acc_rows // NS
    r0 = s * rpt
    pltpu.sync_copy(zeros_hbm.at[pl.ds(r0, rpt)], acc_s.at[pl.ds(r0, rpt)])

    row0 = w * ch
    pltpu.sync_copy(src_hbm.at[pl.ds(row0, ch + 2)], srcblk)
    pltpu.sync_copy(dst_hbm.at[pl.ds(row0, ch + 2)], dstblk)

    plsc.subcore_barrier()

    def _issue(slot, local_row):
        pltpu.async_copy(
            kv_hbm.at[srcblk.at[local_row]], kvbuf.at[slot], sem_kv[slot])
        pltpu.async_copy(
            q_hbm.at[dstblk.at[local_row]], qbuf.at[slot], sem_q[slot])

    def _drain(slot):
        pltpu.make_async_copy(
            kv_hbm.at[srcblk.at[0]], kvbuf.at[slot], sem_kv[slot]).wait()
        pltpu.make_async_copy(
            q_hbm.at[dstblk.at[0]], qbuf.at[slot], sem_q[slot]).wait()

    _issue(0, 0)
    _issue(1, 1)

    def _wait_scatter(b):
        pltpu.make_async_copy(
            wbuf.at[b], acc_s.at[dstblk.at[0]], sem_w[b]).wait()

    def _pair(qq, carry):
        jj = 2 * qq
        for b in range(2):
            _drain(b)

            @pl.when(qq > 0)
            def _():
                _wait_scatter(b)

            _compute_chunk(kvbuf.at[b], qbuf.at[b], wbuf.at[b])
            pltpu.async_copy(
                wbuf.at[b], acc_s.at[dstblk.at[jj + b]], sem_w[b],
                add=True)
            _issue(b, jj + b + 2)
        return carry

    lax.fori_loop(0, ch // 2, _pair, 0)

    _drain(0)
    _drain(1)
    _wait_scatter(0)
    _wait_scatter(1)

    plsc.subcore_barrier()
    pltpu.sync_copy(acc_s.at[pl.ds(r0, rpt)], out_hbm.at[c, pl.ds(r0, rpt)])


def _edge_phase(q, kv, src2, dst2, zeros, acc_rows, ch):
    mesh = plsc.VectorSubcoreMesh(core_axis_name="c", subcore_axis_name="s")
    body = functools.partial(_edge_body, acc_rows, ch)
    return pl.kernel(
        body,
        out_type=jax.ShapeDtypeStruct((NC, acc_rows, ROW), jnp.float32),
        mesh=mesh,
        scratch_types=[
            pltpu.VMEM_SHARED((acc_rows, ROW), jnp.float32),
            pltpu.VMEM((ch + 2, C), jnp.int32),
            pltpu.VMEM((ch + 2, C), jnp.int32),
            pltpu.VMEM((2, C, 2 * HD), jnp.float32),
            pltpu.VMEM((2, C, HD), jnp.float32),
            pltpu.VMEM((2, C, ROW), jnp.float32),
            pltpu.SemaphoreType.DMA,
            pltpu.SemaphoreType.DMA,
            pltpu.SemaphoreType.DMA,
            pltpu.SemaphoreType.DMA,
            pltpu.SemaphoreType.DMA,
            pltpu.SemaphoreType.DMA,
        ],
        compiler_params=pltpu.CompilerParams(
            use_tc_tiling_on_sc=False, needs_layout_passes=False),
    )(q, kv, src2, dst2, zeros)


def _combine_body(acc_ref, out_ref):
    ab = acc_ref[...]
    a = ab[0] + ab[1]
    wv = a[:, :HD]
    z = a[:, HD:HD + H]
    rowi = lax.broadcasted_iota(jnp.int32, (H, HD), 0)
    coli = lax.broadcasted_iota(jnp.int32, (H, HD), 1)
    bmat = (coli // D == rowi).astype(jnp.float32)
    zrep = jnp.dot(z, bmat, preferred_element_type=jnp.float32)
    out_ref[...] = wv / zrep


def _combine(acc, n_out, blk):
    grid = n_out // blk
    return pl.pallas_call(
        _combine_body,
        grid=(grid,),
        in_specs=[pl.BlockSpec((NC, blk, ROW), lambda i: (0, i, 0))],
        out_specs=pl.BlockSpec((blk, HD), lambda i: (i, 0)),
        out_shape=jax.ShapeDtypeStruct((n_out, HD), jnp.float32),
    )(acc)


def kernel(h, edge_index, WQ, WK, WV):
    n, in_dim = h.shape
    e = edge_index.shape[1]

    wt = jnp.concatenate([WQ.T, WK.T, WV.T], axis=1)
    blk_n = 1000 if n % 1000 == 0 else 8
    n_pad = _cdiv(n, blk_n) * blk_n
    h_p = h if n_pad == n else jnp.pad(h, ((0, n_pad - n), (0, 0)))

    src = edge_index[0]
    dst = edge_index[1]
    ch = 2 * _cdiv(e, NW * C * 2)
    rows_total = NW * ch + 2
    e_pad = rows_total * C
    src = jnp.concatenate([src, jnp.zeros((e_pad - e,), jnp.int32)])
    dst = jnp.concatenate([dst, jnp.full((e_pad - e,), n, jnp.int32)])
    src2 = src.reshape(rows_total, C)
    dst2 = dst.reshape(rows_total, C)

    acc_rows = max(n + 1, _cdiv(n, 1000) * 1000)
    acc_rows = NS * _cdiv(acc_rows, NS)
    zeros = jnp.zeros((acc_rows, ROW), jnp.float32)

    q, kv = _qkv(h_p, wt, blk_n)
    q = q[:n] if n_pad != n else q
    kv = kv[:n] if n_pad != n else kv
    acc = _edge_phase(q, kv, src2, dst2, zeros, acc_rows, ch)
    blk_o = 1000 if n % 1000 == 0 else 8
    n_out = _cdiv(n, blk_o) * blk_o
    out = _combine(acc, n_out, blk_o)
    return out[:n].reshape(n, H, D)
